# Initial kernel scaffold; baseline (speedup 1.0000x reference)
import jax
import jax.numpy as jnp
from jax.experimental import pallas as pl


def kernel(x, edge_index, B, Pi):
    raise NotImplementedError("write your pallas kernel here")


# speedup vs baseline: 3.2846x; 3.2846x over previous
import functools

import jax
import jax.numpy as jnp
from jax import lax
from jax.experimental import pallas as pl
from jax.experimental.pallas import tpu as pltpu
from jax.experimental.pallas import tpu_sc as plsc

N_NODES = 50000
C = 20
M = 32
G = 64

_NC = 2
_NS = 16
_NW = _NC * _NS
_CHUNK = 104
_K = 8
_P_PER_W = _CHUNK * _K
_P_PAD = _P_PER_W * _NW
_B_PAD = 2 * _P_PAD


def _table_body(b_ref, pi_ref, t2_ref):
    bv = b_ref[...]
    piv = pi_ref[...]
    eb = jnp.exp(bv - jnp.max(bv, axis=1, keepdims=True))
    sm_b = eb / jnp.sum(eb, axis=1, keepdims=True)
    ep = jnp.exp(piv - jnp.max(piv, axis=0, keepdims=True))
    sm_pi = ep / jnp.sum(ep, axis=0, keepdims=True)
    unnorm = sm_pi[:, None, :] * sm_b + 1e-8
    t = jnp.log(jnp.sum(unnorm, axis=0))
    left = jnp.broadcast_to(t[:, None, :], (M, M, G))
    right = jnp.broadcast_to(t[None, :, :], (M, M, G))
    t2_ref[...] = jnp.concatenate([left, right], axis=-1).reshape(M * M, 2 * G)


def _compute_pair_table(B, Pi):
    return pl.pallas_call(
        _table_body,
        out_shape=jax.ShapeDtypeStruct((M * M, 2 * G), jnp.float32),
    )(B, Pi)


def _gather_body(table_hbm, idx_hbm, out_hbm, idx_v, rows_v, sem):
    wid = lax.axis_index("s") * _NC + lax.axis_index("c")
    pltpu.sync_copy(idx_hbm.at[wid], idx_v)
    copies = [
        pltpu.async_copy(table_hbm.at[idx_v.at[j]], rows_v.at[j], sem)
        for j in range(_K)
    ]
    for cp in copies:
        cp.wait()
    pltpu.sync_copy(rows_v, out_hbm.at[wid])


def _gather(table, idx3d):
    mesh = plsc.VectorSubcoreMesh(core_axis_name="c", subcore_axis_name="s")
    k = pl.kernel(
        _gather_body,
        mesh=mesh,
        out_type=jax.ShapeDtypeStruct((_NW, _K, _CHUNK, 2 * G), jnp.float32),
        scratch_types=[
            pltpu.VMEM((_K, _CHUNK), jnp.int32),
            pltpu.VMEM((_K, _CHUNK, 2 * G), jnp.float32),
            pltpu.SemaphoreType.DMA,
        ],
    )
    return k(table, idx3d)


def kernel(x, edge_index, B, Pi):
    table2 = _compute_pair_table(B, Pi)
    xpad = jnp.pad(x.astype(jnp.int32), (0, _B_PAD - N_NODES))
    xp = xpad.reshape(_P_PAD, 2)
    pair_idx = xp[:, 0] * M + xp[:, 1]
    idx3d = pair_idx.reshape(_NW, _K, _CHUNK)
    rows = _gather(table2, idx3d)
    out = rows.reshape(_B_PAD, G)[:N_NODES]
    return out[:, None, :]


# speedup vs baseline: 5.7312x; 1.7449x over previous
import jax
import jax.numpy as jnp
from jax import lax
from jax.experimental import pallas as pl
from jax.experimental.pallas import tpu as pltpu
from jax.experimental.pallas import tpu_sc as plsc

N_NODES = 50000
C = 20
M = 32
G = 64

N_PAIRS = N_NODES // 2
_NC = 2
_NS = 16
_NW = _NC * _NS
_W = 784
_CHUNKS = (128, 128, 128, 128, 128, 128, 16)


def _table_body(b_ref, pi_ref, t2_ref):
    bv = b_ref[...]
    piv = pi_ref[...]
    eb = jnp.exp(bv - jnp.max(bv, axis=1, keepdims=True))
    sm_b = eb / jnp.sum(eb, axis=1, keepdims=True)
    ep = jnp.exp(piv - jnp.max(piv, axis=0, keepdims=True))
    sm_pi = ep / jnp.sum(ep, axis=0, keepdims=True)
    unnorm = sm_pi[:, None, :] * sm_b + 1e-8
    t = jnp.log(jnp.sum(unnorm, axis=0))
    left = jnp.broadcast_to(t[:, None, :], (M, M, G))
    right = jnp.broadcast_to(t[None, :, :], (M, M, G))
    t2_ref[...] = jnp.concatenate([left, right], axis=-1).reshape(M * M, 2 * G)


def _compute_pair_table(B, Pi):
    return pl.pallas_call(
        _table_body,
        out_shape=jax.ShapeDtypeStruct((M * M, 2 * G), jnp.float32),
    )(B, Pi)


def _gather_body(table_hbm, idx_hbm, out_hbm, idx_v, rows_v, gsem, osem):
    wid = lax.axis_index("s") * _NC + lax.axis_index("c")
    base = jnp.where(wid == _NW - 1, N_PAIRS - _W, wid * _W)
    base = pl.multiple_of(base, 8)
    pltpu.sync_copy(idx_hbm.at[wid], idx_v)
    gathers = []
    off = 0
    for n in _CHUNKS:
        gathers.append((off, n, pltpu.async_copy(
            table_hbm.at[idx_v.at[pl.ds(off, n)]],
            rows_v.at[pl.ds(off, n)], gsem)))
        off += n
    outs = []
    for off, n, cp in gathers:
        cp.wait()
        outs.append(pltpu.async_copy(
            rows_v.at[pl.ds(off, n)],
            out_hbm.at[pl.ds(base + off, n)], osem))
    for cp in outs:
        cp.wait()


def _gather(table, idx2d):
    mesh = plsc.VectorSubcoreMesh(core_axis_name="c", subcore_axis_name="s")
    k = pl.kernel(
        _gather_body,
        mesh=mesh,
        out_type=jax.ShapeDtypeStruct((N_PAIRS, 2 * G), jnp.float32),
        scratch_types=[
            pltpu.VMEM((_W,), jnp.int32),
            pltpu.VMEM((_W, 2 * G), jnp.float32),
            pltpu.SemaphoreType.DMA,
            pltpu.SemaphoreType.DMA,
        ],
    )
    return k(table, idx2d)


def kernel(x, edge_index, B, Pi):
    table2 = _compute_pair_table(B, Pi)
    xp = x.astype(jnp.int32).reshape(N_PAIRS, 2)
    pair_idx = xp[:, 0] * M + xp[:, 1]
    bases = jnp.minimum(jnp.arange(_NW) * _W, N_PAIRS - _W)
    windows = bases[:, None] + jnp.arange(_W)[None, :]
    idx2d = jnp.take(pair_idx, windows, axis=0)
    rows = _gather(table2, idx2d)
    return rows.reshape(N_NODES, G)[:, None, :]


# speedup vs baseline: 7.3539x; 1.2831x over previous
import jax
import jax.numpy as jnp
from jax import lax
from jax.experimental import pallas as pl
from jax.experimental.pallas import tpu as pltpu
from jax.experimental.pallas import tpu_sc as plsc

N_NODES = 50000
C = 20
M = 32
G = 64

N_PAIRS = N_NODES // 2
_NC = 2
_NS = 16
_NW = _NC * _NS
_W = 784
_CHUNKS = (128, 128, 128, 128, 128, 128, 16)


def _table_body(b_ref, pi_ref, t2_ref):
    bv = b_ref[...]
    piv = pi_ref[...]
    eb = jnp.exp(bv - jnp.max(bv, axis=1, keepdims=True))
    sm_b = eb / jnp.sum(eb, axis=1, keepdims=True)
    ep = jnp.exp(piv - jnp.max(piv, axis=0, keepdims=True))
    sm_pi = ep / jnp.sum(ep, axis=0, keepdims=True)
    unnorm = sm_pi[:, None, :] * sm_b + 1e-8
    t = jnp.log(jnp.sum(unnorm, axis=0))
    left = jnp.broadcast_to(t[:, None, :], (M, M, G))
    right = jnp.broadcast_to(t[None, :, :], (M, M, G))
    t2_ref[...] = jnp.concatenate([left, right], axis=-1).reshape(M * M, 2 * G)


def _compute_pair_table(B, Pi):
    return pl.pallas_call(
        _table_body,
        out_shape=jax.ShapeDtypeStruct((M * M, 2 * G), jnp.float32),
    )(B, Pi)


def _gather_body(table_hbm, pidx_hbm, out_hbm, idx_v, rows_v, gsem, osem):
    wid = lax.axis_index("s") * _NC + lax.axis_index("c")
    base = jnp.where(wid == _NW - 1, N_PAIRS - _W, wid * _W)
    base = pl.multiple_of(base, 8)
    pltpu.sync_copy(pidx_hbm.at[pl.ds(base, _W)], idx_v)
    gathers = []
    off = 0
    for n in _CHUNKS:
        gathers.append((off, n, pltpu.async_copy(
            table_hbm.at[idx_v.at[pl.ds(off, n)]],
            rows_v.at[pl.ds(off, n)], gsem)))
        off += n
    outs = []
    for off, n, cp in gathers:
        cp.wait()
        outs.append(pltpu.async_copy(
            rows_v.at[pl.ds(off, n)],
            out_hbm.at[pl.ds(base + off, n)], osem))
    for cp in outs:
        cp.wait()


def _gather(table, pidx):
    mesh = plsc.VectorSubcoreMesh(core_axis_name="c", subcore_axis_name="s")
    k = pl.kernel(
        _gather_body,
        mesh=mesh,
        out_type=jax.ShapeDtypeStruct((N_PAIRS, 2 * G), jnp.float32),
        scratch_types=[
            pltpu.VMEM((_W,), jnp.int32),
            pltpu.VMEM((_W, 2 * G), jnp.float32),
            pltpu.SemaphoreType.DMA,
            pltpu.SemaphoreType.DMA,
        ],
    )
    return k(table, pidx)


def kernel(x, edge_index, B, Pi):
    table2 = _compute_pair_table(B, Pi)
    xi = x.astype(jnp.int32)
    pidx = xi[0::2] * M + xi[1::2]
    rows = _gather(table2, pidx)
    return rows.reshape(N_NODES, G)[:, None, :]


# speedup vs baseline: 8.6082x; 1.1706x over previous
import jax
import jax.numpy as jnp
from jax import lax
from jax.experimental import pallas as pl
from jax.experimental.pallas import tpu as pltpu
from jax.experimental.pallas import tpu_sc as plsc

N_NODES = 50000
C = 20
M = 32
G = 64

N_PAIRS = N_NODES // 2
_NC = 2
_NS = 16
_NW = _NC * _NS
_W = 784
_CHUNKS = (128, 128, 128, 128, 128, 128, 16)


def _table_body(b_ref, pi_ref, t2_ref):
    bv = b_ref[...]
    piv = pi_ref[...]
    eb = jnp.exp(bv - jnp.max(bv, axis=1, keepdims=True))
    sm_b = eb / jnp.sum(eb, axis=1, keepdims=True)
    ep = jnp.exp(piv - jnp.max(piv, axis=0, keepdims=True))
    sm_pi = ep / jnp.sum(ep, axis=0, keepdims=True)
    unnorm = sm_pi[:, None, :] * sm_b + 1e-8
    t = jnp.log(jnp.sum(unnorm, axis=0))
    left = jnp.broadcast_to(t[:, None, :], (M, M, G))
    right = jnp.broadcast_to(t[None, :, :], (M, M, G))
    t2_ref[...] = jnp.concatenate([left, right], axis=-1).reshape(M * M, 2 * G)


def _compute_table(B, Pi):
    return pl.pallas_call(
        _table_body,
        out_shape=jax.ShapeDtypeStruct((M * M, 2 * G), jnp.float32),
    )(B, Pi)


def _gather_body(table_hbm, x_hbm, out_hbm, x_v, idx_v, rows_v, gsem, osem):
    wid = lax.axis_index("s") * _NC + lax.axis_index("c")
    base = jnp.where(wid == _NW - 1, N_PAIRS - _W, wid * _W)
    base = pl.multiple_of(base, 8)
    xoff = pl.multiple_of(base * 2, 16)
    pltpu.sync_copy(x_hbm.at[pl.ds(xoff, 2 * _W)], x_v)
    lane = lax.iota(jnp.int32, 16)
    ev = (lane * 2) % 16
    od = (lane * 2 + 1) % 16
    lo = lane < 8
    dn = lax.GatherDimensionNumbers(
        offset_dims=(), collapsed_slice_dims=(0,), start_index_map=(0,))

    def _shuf(v, i):
        return lax.gather(v, i[:, None], dn, slice_sizes=(1,),
                          mode=lax.GatherScatterMode.PROMISE_IN_BOUNDS)

    for g in range(_W // 16):
        va = x_v[pl.ds(32 * g, 16)]
        vb = x_v[pl.ds(32 * g + 16, 16)]
        even = jnp.where(lo, _shuf(va, ev), _shuf(vb, ev))
        odd = jnp.where(lo, _shuf(va, od), _shuf(vb, od))
        idx_v[pl.ds(16 * g, 16)] = even * M + odd
    gathers = []
    off = 0
    for n in _CHUNKS:
        gathers.append((off, n, pltpu.async_copy(
            table_hbm.at[idx_v.at[pl.ds(off, n)]],
            rows_v.at[pl.ds(off, n)], gsem)))
        off += n
    outs = []
    for off, n, cp in gathers:
        cp.wait()
        outs.append(pltpu.async_copy(
            rows_v.at[pl.ds(off, n)],
            out_hbm.at[pl.ds(base + off, n)], osem))
    for cp in outs:
        cp.wait()


def _gather(table, x):
    mesh = plsc.VectorSubcoreMesh(core_axis_name="c", subcore_axis_name="s")
    k = pl.kernel(
        _gather_body,
        mesh=mesh,
        out_type=jax.ShapeDtypeStruct((N_PAIRS, 2 * G), jnp.float32),
        scratch_types=[
            pltpu.VMEM((2 * _W,), jnp.int32),
            pltpu.VMEM((_W,), jnp.int32),
            pltpu.VMEM((_W, 2 * G), jnp.float32),
            pltpu.SemaphoreType.DMA,
            pltpu.SemaphoreType.DMA,
        ],
    )
    return k(table, x)


def kernel(x, edge_index, B, Pi):
    table2 = _compute_table(B, Pi)
    rows = _gather(table2, x.astype(jnp.int32))
    return rows.reshape(N_NODES, G)[:, None, :]
